# Initial kernel scaffold; baseline (speedup 1.0000x reference)
import jax
import jax.numpy as jnp
from jax.experimental import pallas as pl


def kernel(tk, attention):
    raise NotImplementedError("write your pallas kernel here")


# speedup vs baseline: 428.2409x; 428.2409x over previous
import functools

import jax
import jax.numpy as jnp
from jax.experimental import pallas as pl

_F32 = jnp.float32


def _iota(shape, dim):
    return jax.lax.broadcasted_iota(jnp.int32, shape, dim).astype(_F32)


def _mm(a, b):
    return jax.lax.dot_general(
        a, b, (((1,), (0,)), ((), ())),
        precision=jax.lax.Precision.HIGHEST,
        preferred_element_type=_F32)


def _batch_kernel(tk_ref, attn_ref, out_tok_ref, out_attn_ref):
    T, D = tk_ref.shape[1], tk_ref.shape[2]
    N = T - 1
    dens = N // 2
    Ns = N - dens
    G = Ns // 2

    A_full = attn_ref[0]
    a = A_full[0:1, :]

    aM = jnp.broadcast_to(a, (N, N))
    aT = aM.T
    ii = _iota((N, N), 0)
    jj = _iota((N, N), 1)
    cmp = jnp.where(aT > aM, 1.0, 0.0) + jnp.where((aT == aM) & (ii < jj), 1.0, 0.0)
    rank = jnp.sum(cmp, axis=0, keepdims=True)
    P = jnp.where(ii == rank, 1.0, 0.0)
    ordcol = jnp.sum(P * jj, axis=1, keepdims=True)

    tokens = tk_ref[0, 1:, :]
    P_hi = P[:dens, :]
    P_lo = P[dens:, :]
    ns_tok = _mm(P_hi, tokens)
    skip = _mm(P_lo, tokens)

    A1 = A_full[1:, :]
    t1 = _mm(P_lo, A1)
    skip_attn = _mm(t1, P_lo.T)

    ii2 = _iota((Ns, Ns), 0)
    jj2 = _iota((Ns, Ns), 1)
    adj = skip_attn * jnp.where(ii2 == jj2, 0.0, 1.0)
    m1 = jnp.max(adj, axis=1, keepdims=True)
    j1 = jnp.min(jnp.where(adj == m1, jj2, 1e9), axis=1, keepdims=True)
    sel1 = jj2 == j1
    adj_excl = jnp.where(sel1, -3.0e38, adj)
    m2 = jnp.max(adj_excl, axis=1, keepdims=True)
    j2 = jnp.min(jnp.where(adj_excl == m2, jj2, 1e9), axis=1, keepdims=True)
    self_f = jnp.where(sel1 | (jj2 == j2), 1.0, 0.0)
    binf = jnp.where((self_f + self_f.T) > 0, 1.0, 0.0)
    w = adj * binf
    validf = jnp.where(w != 0, 1.0, 0.0)

    def col_avg(mf):
        deg = jnp.sum(mf, axis=0, keepdims=True)
        s = jnp.sum(w * mf, axis=0, keepdims=True)
        avg = jnp.where(deg > 0, s / deg, 0.0)
        return avg, deg

    avg0, deg0 = col_avg(validf)
    avgM0 = jnp.broadcast_to(avg0, (Ns, Ns))
    degM0 = jnp.broadcast_to(deg0, (Ns, Ns))
    nan_ok = jnp.where((degM0 > 0) & (degM0.T > 0), 1.0, 0.0)
    cur0 = validf * nan_ok * jnp.where(avgM0 > avgM0.T, 1.0, 0.0)

    def body(st):
        _, cur = st
        avg, _ = col_avg(cur)
        avgM = jnp.broadcast_to(avg, (Ns, Ns))
        new = cur * jnp.where(avgM > avgM.T, 1.0, 0.0)
        return (cur, new)

    st = body((validf, cur0))
    st = jax.lax.while_loop(lambda s: jnp.sum(s[1]) != jnp.sum(s[0]), body, st)
    mf = st[0]

    out_embs = _mm(mf.T, skip) + skip
    nd = jnp.sum(mf, axis=0, keepdims=True)
    ndM = jnp.broadcast_to(nd, (Ns, Ns))
    gcmp = jnp.where(ndM.T > ndM, 1.0, 0.0) + jnp.where((ndM.T == ndM) & (ii2 < jj2), 1.0, 0.0)
    grank = jnp.sum(gcmp, axis=0, keepdims=True)
    Gmat = jnp.where(_iota((G, Ns), 0) == grank, 1.0, 0.0)
    g_embs = _mm(Gmat, out_embs)
    ord_hi = _mm(Gmat, ordcol[dens:, :])

    cls_tok = tk_ref[0, 0:1, :]
    out_tok_ref[0] = jnp.concatenate([cls_tok, ns_tok, g_embs], axis=0)

    AR = jnp.concatenate([a, _mm(P_hi, A1), _mm(Gmat, t1)], axis=0)
    Ccol = jnp.concatenate([ordcol[:dens, :], ord_hi], axis=0)
    vv = _iota((dens + G, N), 1)
    Csel = jnp.where(vv == Ccol, 1.0, 0.0)
    out_attn_ref[0] = _mm(AR, Csel.T)


def kernel(tk, attention):
    B, T, D = tk.shape
    N = T - 1
    dens = N // 2
    G = (N - dens) // 2
    To = dens + G + 1
    No = dens + G
    out_shape = (
        jax.ShapeDtypeStruct((B, To, D), tk.dtype),
        jax.ShapeDtypeStruct((B, To, No), attention.dtype),
    )
    return pl.pallas_call(
        _batch_kernel,
        grid=(B,),
        in_specs=[
            pl.BlockSpec((1, T, D), lambda b: (b, 0, 0)),
            pl.BlockSpec((1, T, N), lambda b: (b, 0, 0)),
        ],
        out_specs=(
            pl.BlockSpec((1, To, D), lambda b: (b, 0, 0)),
            pl.BlockSpec((1, To, No), lambda b: (b, 0, 0)),
        ),
        out_shape=out_shape,
    )(tk, attention)
